# baseline (speedup 1.0000x reference)
import jax
import jax.numpy as jnp
from jax import lax
from jax.experimental import pallas as pl
from jax.experimental.pallas import tpu as pltpu
from jax.experimental.pallas import tpu_sc as plsc

_NBUF = 2
_NW = 32
_L = 16


def kernel(token_ids, weight):
    b, t = token_ids.shape
    nv, d = weight.shape
    bb = b // _NW
    assert b % _NW == 0 and bb % 8 == 0 and d % _L == 0
    idxT = token_ids.T.astype(jnp.int32)

    mesh = plsc.VectorSubcoreMesh(core_axis_name="core",
                                  subcore_axis_name="subcore")

    scratch = ([pltpu.VMEM((t, bb), jnp.int32)]
               + [pltpu.VMEM((bb, d), jnp.float32) for _ in range(_NBUF)]
               + [pltpu.VMEM((d, bb), jnp.float32) for _ in range(_NBUF)]
               + [pltpu.SemaphoreType.DMA for _ in range(2 * _NBUF)])

    @pl.kernel(out_type=jax.ShapeDtypeStruct((t, d, b), weight.dtype),
               mesh=mesh,
               compiler_params=pltpu.CompilerParams(
                   use_tc_tiling_on_sc=False, needs_layout_passes=False,
                   disable_bounds_checks=True),
               scratch_types=scratch)
    def gather_kernel(table_hbm, idx_hbm, x_hbm, idx_v, *rest):
        bufs = rest[:_NBUF]
        xbufs = rest[_NBUF:2 * _NBUF]
        gsem = rest[2 * _NBUF:3 * _NBUF]
        wsem = rest[3 * _NBUF:]
        wid = lax.axis_index("subcore") * 2 + lax.axis_index("core")
        base = wid * bb
        pltpu.sync_copy(idx_hbm.at[:, pl.ds(base, bb)], idx_v)

        def start_gather(k, tt):
            pltpu.make_async_copy(
                table_hbm.at[idx_v.at[tt]], bufs[k], gsem[k]).start()

        def wait_gather(k):
            pltpu.make_async_copy(
                table_hbm.at[idx_v.at[0]], bufs[k], gsem[k]).wait()

        def start_wb(k, tt):
            pltpu.make_async_copy(
                xbufs[k], x_hbm.at[tt, :, pl.ds(base, bb)], wsem[k]).start()

        def wait_wb(k):
            pltpu.make_async_copy(
                xbufs[k], x_hbm.at[0, :, pl.ds(base, bb)], wsem[k]).wait()

        def transpose(k):
            @pl.loop(0, bb, step=_L)
            def _(c0):
                rows = lax.broadcasted_iota(jnp.int32, (_L,), 0) + c0
                for j in range(d):
                    cols = jnp.full((_L,), j, jnp.int32)
                    xbufs[k][j, pl.ds(c0, _L)] = plsc.load_gather(
                        bufs[k], [rows, cols])

        for k in range(_NBUF):
            start_gather(k, k)

        @pl.loop(0, t, step=_NBUF)
        def _(g):
            for k in range(_NBUF):
                tt = g + k
                wait_gather(k)
                transpose(k)
                start_wb(k, tt)
                wait_wb(k)

                @pl.when(tt + _NBUF < t)
                def _():
                    start_gather(k, tt + _NBUF)

    return gather_kernel(weight, idxT).transpose(2, 0, 1)
